# baseline (speedup 1.0000x reference)
import functools

import jax
import jax.numpy as jnp
from jax import lax
from jax.experimental import pallas as pl
from jax.experimental.pallas import tpu as pltpu
from jax.experimental.pallas import tpu_sc as plsc

_NC, _NS, _L = 2, 16, 16
_NW = _NC * _NS
_GRP = 16
_NBUF = 4


def _make_sc_kernel(N, D2, MD, MS, n_per_w):
    D = 2 * D2
    n_groups = n_per_w // _GRP
    g_elems = _GRP * D
    tab_elems = (MD + MS) * D2
    mesh = plsc.VectorSubcoreMesh(core_axis_name="c", subcore_axis_name="s")

    @functools.partial(
        pl.kernel,
        mesh=mesh,
        out_type=jax.ShapeDtypeStruct((N * D,), jnp.float32),
        scratch_types=[
            pltpu.VMEM((tab_elems,), jnp.float32),
            pltpu.VMEM((n_per_w,), jnp.int32),
            pltpu.VMEM((n_per_w,), jnp.int32),
            pltpu.VMEM((_NBUF * g_elems,), jnp.float32),
            pltpu.SemaphoreType.DMA,
        ],
    )
    def k(dep_hbm, sib_hbm, d_hbm, s_hbm, out_hbm, tab, dv, sv, obuf, osem):
        wid = lax.axis_index("s") * _NC + lax.axis_index("c")
        tok0 = wid * n_per_w

        pltpu.sync_copy(dep_hbm, tab.at[pl.ds(0, MD * D2)])
        pltpu.sync_copy(sib_hbm, tab.at[pl.ds(MD * D2, MS * D2)])
        pltpu.sync_copy(d_hbm.at[pl.ds(tok0, n_per_w)], dv)
        pltpu.sync_copy(s_hbm.at[pl.ds(tok0, n_per_w)], sv)

        lane = lax.iota(jnp.int32, _L)

        def do_group(g):
            slot = lax.rem(g, _NBUF)
            sbase = slot * g_elems
            dvec = jnp.clip(dv[pl.ds(g * _GRP, _GRP)], 0, MD - 1) * D2
            svec = (jnp.clip(sv[pl.ds(g * _GRP, _GRP)], 0, MS - 1) + MD) * D2
            for j in range(_GRP):
                db = dvec[j]
                sb = svec[j]
                obase = sbase + j * D
                for u in range(D2 // _L):
                    o = u * _L
                    obuf[pl.ds(obase + o, _L)] = tab[pl.ds(db + o, _L)]
                for u in range(D2 // _L):
                    o = u * _L
                    obuf[pl.ds(obase + D2 + o, _L)] = tab[pl.ds(sb + o, _L)]
            off = pl.multiple_of((tok0 + g * _GRP) * D, 8)
            return pltpu.async_copy(
                obuf.at[pl.ds(sbase, g_elems)],
                out_hbm.at[pl.ds(off, g_elems)], osem)

        def drain_one(g):
            slot = lax.rem(g, _NBUF)
            pltpu.make_async_copy(
                out_hbm.at[pl.ds(0, g_elems)],
                obuf.at[pl.ds(slot * g_elems, g_elems)], osem).wait()

        def body(g, c):
            @pl.when(g >= _NBUF)
            def _():
                drain_one(g - _NBUF)
            do_group(g)
            return c
        lax.fori_loop(0, n_groups, body, 0)

        for g in range(n_groups - _NBUF, n_groups):
            drain_one(g)

    return k


def kernel(seq_len, tree_depths, sibling_indices, depth_embedding,
           sibling_embedding, pos_embedding):
    B, S = tree_depths.shape
    N = B * S
    MD, D2 = depth_embedding.shape
    MS = sibling_embedding.shape[0]
    n_per_w = N // _NW

    dep_flat = depth_embedding.reshape(MD * D2)
    sib_flat = sibling_embedding.reshape(MS * D2)
    d_flat = tree_depths.reshape(N)
    s_flat = sibling_indices.reshape(N)

    k = _make_sc_kernel(N, D2, MD, MS, n_per_w)
    out = k(dep_flat, sib_flat, d_flat, s_flat)
    return out.reshape(B, S, 2 * D2)


# speedup vs baseline: 2.5549x; 2.5549x over previous
import functools

import jax
import jax.numpy as jnp
from jax import lax
from jax.experimental import pallas as pl
from jax.experimental.pallas import tpu as pltpu
from jax.experimental.pallas import tpu_sc as plsc

_NC, _NS, _L = 2, 16, 16
_NW = _NC * _NS
_GROUP = 16
_NBUF = 4


def _build_fused_table(dep, sib):
    MD, D2 = dep.shape
    MS = sib.shape[0]

    def body(dep_ref, sib_ref, out_ref):
        d = jnp.broadcast_to(dep_ref[...][:, None, :], (MD, MS, D2))
        s = jnp.broadcast_to(sib_ref[...][None, :, :], (MD, MS, D2))
        out_ref[...] = jnp.concatenate([d, s], axis=2).reshape(MD * MS, 2 * D2)

    return pl.pallas_call(
        body,
        out_shape=jax.ShapeDtypeStruct((MD * MS, 2 * D2), jnp.float32),
    )(dep, sib)


def _make_sc_kernel(N, D2, MD, MS, n_per_w, n_groups):
    D = 2 * D2
    mesh = plsc.VectorSubcoreMesh(core_axis_name="c", subcore_axis_name="s")

    @functools.partial(
        pl.kernel,
        mesh=mesh,
        out_type=jax.ShapeDtypeStruct((N, D), jnp.float32),
        scratch_types=[
            pltpu.VMEM((n_per_w,), jnp.int32),
            pltpu.VMEM((n_per_w,), jnp.int32),
            pltpu.VMEM((n_per_w,), jnp.int32),
            pltpu.VMEM((_NBUF, _GROUP, D), jnp.float32),
            pltpu.SemaphoreType.DMA,
            pltpu.SemaphoreType.DMA,
            pltpu.SemaphoreType.DMA,
        ],
    )
    def k(tab_hbm, d_hbm, s_hbm, out_hbm, dv, sv, iv, gbuf, gsem, gsem2, osem):
        wid = lax.axis_index("s") * _NC + lax.axis_index("c")
        tok0 = wid * n_per_w

        pltpu.sync_copy(d_hbm.at[pl.ds(tok0, n_per_w)], dv)
        pltpu.sync_copy(s_hbm.at[pl.ds(tok0, n_per_w)], sv)

        for c in range(n_per_w // _L):
            d = jnp.clip(dv[pl.ds(c * _L, _L)], 0, MD - 1)
            s = jnp.clip(sv[pl.ds(c * _L, _L)], 0, MS - 1)
            iv[pl.ds(c * _L, _L)] = d * MS + s

        def gather(g, slot):
            idx = iv.at[pl.ds(g * _GROUP, _GROUP)]
            sem = gsem if g % 2 == 0 else gsem2
            return pltpu.async_copy(tab_hbm.at[idx], gbuf.at[slot], sem)

        def put(g, slot):
            off = pl.multiple_of(tok0 + g * _GROUP, 8)
            dst = out_hbm.at[pl.ds(off, _GROUP)]
            return pltpu.async_copy(gbuf.at[slot], dst, osem)

        gh, oh = {}, {}
        unwaited = set()
        for g in range(min(_NBUF - 1, n_groups)):
            gh[g] = gather(g, g % _NBUF)
        for g in range(n_groups):
            gh[g].wait()
            oh[g] = put(g, g % _NBUF)
            unwaited.add(g)
            ng = g + _NBUF - 1
            if ng < n_groups:
                prev = ng - _NBUF
                if prev >= 0:
                    oh[prev].wait()
                    unwaited.discard(prev)
                gh[ng] = gather(ng, ng % _NBUF)
        for g in sorted(unwaited):
            oh[g].wait()

    return k


def kernel(seq_len, tree_depths, sibling_indices, depth_embedding,
           sibling_embedding, pos_embedding):
    B, S = tree_depths.shape
    N = B * S
    MD, D2 = depth_embedding.shape
    MS = sibling_embedding.shape[0]
    n_per_w = N // _NW
    n_groups = n_per_w // _GROUP

    tab = _build_fused_table(depth_embedding, sibling_embedding)
    d_flat = tree_depths.reshape(N)
    s_flat = sibling_indices.reshape(N)

    k = _make_sc_kernel(N, D2, MD, MS, n_per_w, n_groups)
    out = k(tab, d_flat, s_flat)
    return out.reshape(B, S, 2 * D2)


# speedup vs baseline: 2.8390x; 1.1112x over previous
import functools

import jax
import jax.numpy as jnp
from jax import lax
from jax.experimental import pallas as pl
from jax.experimental.pallas import tpu as pltpu
from jax.experimental.pallas import tpu_sc as plsc

_NC, _NS, _L = 2, 16, 16
_NW = _NC * _NS
_GG = 8
_GPB = 3
_CG = 8
_NBODY = 32


def _build_fused_table(dep, sib):
    MD, D2 = dep.shape
    MS = sib.shape[0]

    def body(dep_ref, sib_ref, out_ref):
        d = jnp.broadcast_to(dep_ref[...][:, None, :], (MD, MS, D2))
        s = jnp.broadcast_to(sib_ref[...][None, :, :], (MD, MS, D2))
        out_ref[...] = jnp.concatenate([d, s], axis=2).reshape(MD * MS, 2 * D2)

    return pl.pallas_call(
        body,
        out_shape=jax.ShapeDtypeStruct((MD * MS, 2 * D2), jnp.float32),
    )(dep, sib)


def _make_sc_kernel(N, D2, MD, MS, n_per_w):
    D = 2 * D2
    n_gtok = _NBODY * _GPB * _GG
    g_elems = _GG * D
    c_elems = _CG * D
    tab_elems = (MD + MS) * D2
    mesh = plsc.VectorSubcoreMesh(core_axis_name="c", subcore_axis_name="s")

    @functools.partial(
        pl.kernel,
        mesh=mesh,
        out_type=jax.ShapeDtypeStruct((N, D), jnp.float32),
        scratch_types=[
            pltpu.VMEM((tab_elems,), jnp.float32),
            pltpu.VMEM((n_per_w + _L,), jnp.int32),
            pltpu.VMEM((n_per_w + _L,), jnp.int32),
            pltpu.VMEM((n_gtok,), jnp.int32),
            pltpu.VMEM((6, _GG, D), jnp.float32),
            pltpu.VMEM((2, _CG, D), jnp.float32),
            pltpu.SemaphoreType.DMA,
            pltpu.SemaphoreType.DMA,
            pltpu.SemaphoreType.DMA,
        ],
    )
    def k(tab_hbm, dep_hbm, sib_hbm, d_hbm, s_hbm, out_hbm,
          tab, dv, sv, iv, gbuf, cbuf, gsem, osem, csem):
        wid = lax.axis_index("s") * _NC + lax.axis_index("c")
        tok0 = wid * n_per_w

        pltpu.sync_copy(dep_hbm, tab.at[pl.ds(0, MD * D2)])
        pltpu.sync_copy(sib_hbm, tab.at[pl.ds(MD * D2, MS * D2)])
        pltpu.sync_copy(d_hbm.at[pl.ds(tok0, n_per_w)], dv.at[pl.ds(0, n_per_w)])
        pltpu.sync_copy(s_hbm.at[pl.ds(tok0, n_per_w)], sv.at[pl.ds(0, n_per_w)])

        for c in range(n_gtok // _L):
            d = jnp.clip(dv[pl.ds(c * _L, _L)], 0, MD - 1)
            s = jnp.clip(sv[pl.ds(c * _L, _L)], 0, MS - 1)
            iv[pl.ds(c * _L, _L)] = d * MS + s

        def gather(g):
            slot = lax.rem(g, 6)
            idx = iv.at[pl.ds(g * _GG, _GG)]
            return pltpu.async_copy(tab_hbm.at[idx], gbuf.at[slot], gsem)

        def put_gather(g):
            slot = lax.rem(g, 6)
            off = pl.multiple_of(tok0 + g * _GG, 8)
            return pltpu.async_copy(
                gbuf.at[slot], out_hbm.at[pl.ds(off, _GG)], osem)

        def drain(sem, ref, rows):
            pltpu.make_async_copy(
                out_hbm.at[pl.ds(0, rows)], ref.at[0], sem).wait()

        def copy_group(i):
            base = n_gtok + i * _CG
            slot = lax.rem(i, 2)
            dvec = jnp.clip(dv[pl.ds(base, _L)], 0, MD - 1) * D2
            svec = (jnp.clip(sv[pl.ds(base, _L)], 0, MS - 1) + MD) * D2
            for j in range(_CG):
                db = dvec[j]
                sb = svec[j]
                for u in range(D2 // _L):
                    o = u * _L
                    cbuf[slot, j, pl.ds(o, _L)] = tab[pl.ds(db + o, _L)]
                for u in range(D2 // _L):
                    o = u * _L
                    cbuf[slot, j, pl.ds(D2 + o, _L)] = tab[pl.ds(sb + o, _L)]
            off = pl.multiple_of(tok0 + base, 8)
            return pltpu.async_copy(
                cbuf.at[slot], out_hbm.at[pl.ds(off, _CG)], csem)

        def body(i, carry):
            @pl.when(i >= 2)
            def _():
                for _k in range(_GPB):
                    drain(osem, gbuf, _GG)
                drain(csem, cbuf, _CG)
            for _k in range(_GPB):
                gather(i * _GPB + _k)
            copy_group(i)
            @pl.when(i >= 1)
            def _():
                for _k in range(_GPB):
                    drain(gsem, gbuf, _GG)
                for _k in range(_GPB):
                    put_gather((i - 1) * _GPB + _k)
            return carry

        lax.fori_loop(0, _NBODY, body, 0)

        for _k in range(_GPB):
            drain(gsem, gbuf, _GG)
        for _k in range(_GPB):
            put_gather((_NBODY - 1) * _GPB + _k)
        for _k in range(2 * _GPB):
            drain(osem, gbuf, _GG)
        for _k in range(2):
            drain(csem, cbuf, _CG)

    return k


def kernel(seq_len, tree_depths, sibling_indices, depth_embedding,
           sibling_embedding, pos_embedding):
    B, S = tree_depths.shape
    N = B * S
    MD, D2 = depth_embedding.shape
    MS = sibling_embedding.shape[0]
    n_per_w = N // _NW

    tab = _build_fused_table(depth_embedding, sibling_embedding)
    dep_flat = depth_embedding.reshape(MD * D2)
    sib_flat = sibling_embedding.reshape(MS * D2)
    d_flat = tree_depths.reshape(N)
    s_flat = sibling_indices.reshape(N)

    k = _make_sc_kernel(N, D2, MD, MS, n_per_w)
    out = k(tab, dep_flat, sib_flat, d_flat, s_flat)
    return out.reshape(B, S, 2 * D2)


# speedup vs baseline: 2.8490x; 1.0035x over previous
import functools

import jax
import jax.numpy as jnp
from jax import lax
from jax.experimental import pallas as pl
from jax.experimental.pallas import tpu as pltpu
from jax.experimental.pallas import tpu_sc as plsc

_NC, _NS, _L = 2, 16, 16
_NW = _NC * _NS
_GG = 8
_GPB = 3
_CG = 8
_NBODY = 32


def _build_fused_table(dep, sib):
    MD, D2 = dep.shape
    MS = sib.shape[0]

    def body(dep_ref, sib_ref, out_ref):
        d = jnp.broadcast_to(dep_ref[...][:, None, :], (MD, MS, D2))
        s = jnp.broadcast_to(sib_ref[...][None, :, :], (MD, MS, D2))
        out_ref[...] = jnp.concatenate([d, s], axis=2).reshape(MD * MS, 2 * D2)

    return pl.pallas_call(
        body,
        out_shape=jax.ShapeDtypeStruct((MD * MS, 2 * D2), jnp.float32),
    )(dep, sib)


def _make_sc_kernel(N, D2, MD, MS, n_per_w):
    D = 2 * D2
    n_gtok = _NBODY * _GPB * _GG
    g_elems = _GG * D
    c_elems = _CG * D
    tab_elems = (MD + MS) * D2
    mesh = plsc.VectorSubcoreMesh(core_axis_name="c", subcore_axis_name="s")

    @functools.partial(
        pl.kernel,
        mesh=mesh,
        out_type=jax.ShapeDtypeStruct((N, D), jnp.float32),
        scratch_types=[
            pltpu.VMEM((tab_elems,), jnp.float32),
            pltpu.VMEM((n_per_w + _L,), jnp.int32),
            pltpu.VMEM((n_per_w + _L,), jnp.int32),
            pltpu.VMEM((n_gtok,), jnp.int32),
            pltpu.VMEM((9, _GG, D), jnp.float32),
            pltpu.VMEM((2, _CG, D), jnp.float32),
            pltpu.SemaphoreType.DMA,
            pltpu.SemaphoreType.DMA,
            pltpu.SemaphoreType.DMA,
        ],
    )
    def k(tab_hbm, dep_hbm, sib_hbm, d_hbm, s_hbm, out_hbm,
          tab, dv, sv, iv, gbuf, cbuf, gsem, osem, csem):
        wid = lax.axis_index("s") * _NC + lax.axis_index("c")
        tok0 = wid * n_per_w

        pltpu.sync_copy(dep_hbm, tab.at[pl.ds(0, MD * D2)])
        pltpu.sync_copy(sib_hbm, tab.at[pl.ds(MD * D2, MS * D2)])
        pltpu.sync_copy(d_hbm.at[pl.ds(tok0, n_per_w)], dv.at[pl.ds(0, n_per_w)])
        pltpu.sync_copy(s_hbm.at[pl.ds(tok0, n_per_w)], sv.at[pl.ds(0, n_per_w)])

        for c in range(n_gtok // _L):
            d = jnp.clip(dv[pl.ds(c * _L, _L)], 0, MD - 1)
            s = jnp.clip(sv[pl.ds(c * _L, _L)], 0, MS - 1)
            iv[pl.ds(c * _L, _L)] = d * MS + s

        def gather(g):
            slot = lax.rem(g, 9)
            idx = iv.at[pl.ds(g * _GG, _GG)]
            return pltpu.async_copy(tab_hbm.at[idx], gbuf.at[slot], gsem)

        def put_gather(g):
            slot = lax.rem(g, 9)
            off = pl.multiple_of(tok0 + g * _GG, 8)
            return pltpu.async_copy(
                gbuf.at[slot], out_hbm.at[pl.ds(off, _GG)], osem)

        def drain(sem, ref, rows):
            pltpu.make_async_copy(
                out_hbm.at[pl.ds(0, rows)], ref.at[0], sem).wait()

        def copy_group(i):
            base = n_gtok + i * _CG
            slot = lax.rem(i, 2)
            dvec = jnp.clip(dv[pl.ds(base, _L)], 0, MD - 1) * D2
            svec = (jnp.clip(sv[pl.ds(base, _L)], 0, MS - 1) + MD) * D2
            for j in range(_CG):
                db = dvec[j]
                sb = svec[j]
                for u in range(D2 // _L):
                    o = u * _L
                    cbuf[slot, j, pl.ds(o, _L)] = tab[pl.ds(db + o, _L)]
                for u in range(D2 // _L):
                    o = u * _L
                    cbuf[slot, j, pl.ds(D2 + o, _L)] = tab[pl.ds(sb + o, _L)]
            off = pl.multiple_of(tok0 + base, 8)
            return pltpu.async_copy(
                cbuf.at[slot], out_hbm.at[pl.ds(off, _CG)], csem)

        def body(i, carry):
            @pl.when(i >= 3)
            def _():
                for _k in range(_GPB):
                    drain(osem, gbuf, _GG)
            @pl.when(i >= 2)
            def _():
                drain(csem, cbuf, _CG)
            for _k in range(_GPB):
                gather(i * _GPB + _k)
            copy_group(i)
            @pl.when(i >= 2)
            def _():
                for _k in range(_GPB):
                    drain(gsem, gbuf, _GG)
                for _k in range(_GPB):
                    put_gather((i - 2) * _GPB + _k)
            return carry

        lax.fori_loop(0, _NBODY, body, 0)

        for b in (_NBODY - 2, _NBODY - 1):
            for _k in range(_GPB):
                drain(gsem, gbuf, _GG)
            for _k in range(_GPB):
                put_gather(b * _GPB + _k)
        for _k in range(3 * _GPB):
            drain(osem, gbuf, _GG)
        for _k in range(2):
            drain(csem, cbuf, _CG)

    return k


def kernel(seq_len, tree_depths, sibling_indices, depth_embedding,
           sibling_embedding, pos_embedding):
    B, S = tree_depths.shape
    N = B * S
    MD, D2 = depth_embedding.shape
    MS = sibling_embedding.shape[0]
    n_per_w = N // _NW

    tab = _build_fused_table(depth_embedding, sibling_embedding)
    dep_flat = depth_embedding.reshape(MD * D2)
    sib_flat = sibling_embedding.reshape(MS * D2)
    d_flat = tree_depths.reshape(N)
    s_flat = sibling_indices.reshape(N)

    k = _make_sc_kernel(N, D2, MD, MS, n_per_w)
    out = k(tab, dep_flat, sib_flat, d_flat, s_flat)
    return out.reshape(B, S, 2 * D2)


# speedup vs baseline: 2.8622x; 1.0046x over previous
import functools

import jax
import jax.numpy as jnp
from jax import lax
from jax.experimental import pallas as pl
from jax.experimental.pallas import tpu as pltpu
from jax.experimental.pallas import tpu_sc as plsc

_NC, _NS, _L = 2, 16, 16
_NW = _NC * _NS
_GG = 8
_GPB = 3
_CG = 8
_NBODY = 32


def _build_fused_table(dep, sib):
    MD, D2 = dep.shape
    MS = sib.shape[0]

    def body(dep_ref, sib_ref, out_ref):
        d = jnp.broadcast_to(dep_ref[...][:, None, :], (MD, MS, D2))
        s = jnp.broadcast_to(sib_ref[...][None, :, :], (MD, MS, D2))
        out_ref[...] = jnp.concatenate([d, s], axis=2).reshape(MD * MS, 2 * D2)

    return pl.pallas_call(
        body,
        out_shape=jax.ShapeDtypeStruct((MD * MS, 2 * D2), jnp.float32),
    )(dep, sib)


def _make_sc_kernel(N, D2, MD, MS, n_per_w):
    D = 2 * D2
    n_gtok = _NBODY * _GPB * _GG
    g_elems = _GG * D
    c_elems = _CG * D
    tab_elems = (MD + MS) * D2
    mesh = plsc.VectorSubcoreMesh(core_axis_name="c", subcore_axis_name="s")

    @functools.partial(
        pl.kernel,
        mesh=mesh,
        out_type=jax.ShapeDtypeStruct((N, D), jnp.float32),
        scratch_types=[
            pltpu.VMEM((tab_elems,), jnp.float32),
            pltpu.VMEM((n_per_w + _L,), jnp.int32),
            pltpu.VMEM((n_per_w + _L,), jnp.int32),
            pltpu.VMEM((n_gtok,), jnp.int32),
            pltpu.VMEM((9, _GG, D), jnp.float32),
            pltpu.VMEM((4, _CG, D), jnp.float32),
            pltpu.SemaphoreType.DMA,
            pltpu.SemaphoreType.DMA,
            pltpu.SemaphoreType.DMA,
        ],
    )
    def k(tab_hbm, dep_hbm, sib_hbm, d_hbm, s_hbm, out_hbm,
          tab, dv, sv, iv, gbuf, cbuf, gsem, osem, csem):
        wid = lax.axis_index("s") * _NC + lax.axis_index("c")
        tok0 = wid * n_per_w

        pltpu.sync_copy(dep_hbm, tab.at[pl.ds(0, MD * D2)])
        pltpu.sync_copy(sib_hbm, tab.at[pl.ds(MD * D2, MS * D2)])
        pltpu.sync_copy(d_hbm.at[pl.ds(tok0, n_per_w)], dv.at[pl.ds(0, n_per_w)])
        pltpu.sync_copy(s_hbm.at[pl.ds(tok0, n_per_w)], sv.at[pl.ds(0, n_per_w)])

        def build_idx(c):
            d = jnp.clip(dv[pl.ds(c * _L, _L)], 0, MD - 1)
            s = jnp.clip(sv[pl.ds(c * _L, _L)], 0, MS - 1)
            iv[pl.ds(c * _L, _L)] = d * MS + s

        def gather(g):
            slot = lax.rem(g, 9)
            idx = iv.at[pl.ds(g * _GG, _GG)]
            return pltpu.async_copy(tab_hbm.at[idx], gbuf.at[slot], gsem)

        def put_gather(g):
            slot = lax.rem(g, 9)
            off = pl.multiple_of(tok0 + g * _GG, 8)
            return pltpu.async_copy(
                gbuf.at[slot], out_hbm.at[pl.ds(off, _GG)], osem)

        def drain(sem, ref, rows):
            pltpu.make_async_copy(
                out_hbm.at[pl.ds(0, rows)], ref.at[0], sem).wait()

        def copy_group(i):
            base = n_gtok + i * _CG
            slot = lax.rem(i, 4)
            dvec = jnp.clip(dv[pl.ds(base, _L)], 0, MD - 1) * D2
            svec = (jnp.clip(sv[pl.ds(base, _L)], 0, MS - 1) + MD) * D2
            for j in range(_CG):
                db = dvec[j]
                sb = svec[j]
                for u in range(D2 // _L):
                    o = u * _L
                    cbuf[slot, j, pl.ds(o, _L)] = tab[pl.ds(db + o, _L)]
                for u in range(D2 // _L):
                    o = u * _L
                    cbuf[slot, j, pl.ds(D2 + o, _L)] = tab[pl.ds(sb + o, _L)]
            off = pl.multiple_of(tok0 + base, 8)
            return pltpu.async_copy(
                cbuf.at[slot], out_hbm.at[pl.ds(off, _CG)], csem)

        def build_idx2(i):
            build_idx(2 * i)
            build_idx(2 * i + 1)

        def body(i, carry):
            @pl.when(i >= 3)
            def _():
                for _k in range(_GPB):
                    drain(osem, gbuf, _GG)
            @pl.when(i >= 4)
            def _():
                drain(csem, cbuf, _CG)
            @pl.when(i < (n_gtok // _L) // 2)
            def _():
                build_idx2(i)
            for _k in range(_GPB):
                gather(i * _GPB + _k)
            copy_group(i)
            @pl.when(i >= 2)
            def _():
                for _k in range(_GPB):
                    drain(gsem, gbuf, _GG)
                for _k in range(_GPB):
                    put_gather((i - 2) * _GPB + _k)
            return carry

        lax.fori_loop(0, _NBODY, body, 0)

        for b in (_NBODY - 2, _NBODY - 1):
            for _k in range(_GPB):
                drain(gsem, gbuf, _GG)
            for _k in range(_GPB):
                put_gather(b * _GPB + _k)
        for _k in range(3 * _GPB):
            drain(osem, gbuf, _GG)
        for _k in range(4):
            drain(csem, cbuf, _CG)

    return k


def kernel(seq_len, tree_depths, sibling_indices, depth_embedding,
           sibling_embedding, pos_embedding):
    B, S = tree_depths.shape
    N = B * S
    MD, D2 = depth_embedding.shape
    MS = sibling_embedding.shape[0]
    n_per_w = N // _NW

    tab = _build_fused_table(depth_embedding, sibling_embedding)
    dep_flat = depth_embedding.reshape(MD * D2)
    sib_flat = sibling_embedding.reshape(MS * D2)
    d_flat = tree_depths.reshape(N)
    s_flat = sibling_indices.reshape(N)

    k = _make_sc_kernel(N, D2, MD, MS, n_per_w)
    out = k(tab, dep_flat, sib_flat, d_flat, s_flat)
    return out.reshape(B, S, 2 * D2)
